# baseline (speedup 1.0000x reference)
import functools

import jax
import jax.numpy as jnp
from jax import lax
from jax.experimental import pallas as pl
from jax.experimental.pallas import tpu as pltpu
from jax.experimental.pallas import tpu_sc as plsc

EMB = 64
NE = 1024
NT = 36864
GAMMA = 0.99

BLK = 1024
NBLK = NT // BLK

NC = 2
NS = 16
NW = NC * NS
ROWS_PER_W = NT // NW
CHUNK = 128
NCH = ROWS_PER_W // CHUNK
NROWCH = NT // CHUNK
SC_ROWS = NE // NS


def _dist_argmin_body(x_ref, vqt_ref, idx_ref, counts_ref):
    i = pl.program_id(0)
    xb = x_ref[...]
    vt = vqt_ref[...]
    dot = lax.dot_general(xb, vt, (((1,), (0,)), ((), ())),
                          preferred_element_type=jnp.float32)
    xs = jnp.sum(xb * xb, axis=1, keepdims=True)
    vs = jnp.sum(vt * vt, axis=0, keepdims=True)
    d = xs - 2.0 * dot + vs
    m = jnp.min(d, axis=1, keepdims=True)
    iota = lax.broadcasted_iota(jnp.int32, d.shape, 1)
    idx = jnp.min(jnp.where(d <= m, iota, jnp.int32(NE)), axis=1)
    idx_ref[0, 0, :] = idx
    partial = jnp.sum((iota == idx[:, None]).astype(jnp.float32),
                      axis=0, keepdims=True)

    @pl.when(i == 0)
    def _():
        counts_ref[...] = jnp.zeros_like(counts_ref)

    counts_ref[...] += partial


def _dist_argmin(x, vqt):
    return pl.pallas_call(
        _dist_argmin_body,
        grid=(NBLK,),
        in_specs=[
            pl.BlockSpec((BLK, EMB), lambda i: (i, 0)),
            pl.BlockSpec((EMB, NE), lambda i: (0, 0)),
        ],
        out_specs=[
            pl.BlockSpec((1, 1, BLK), lambda i: (i, 0, 0)),
            pl.BlockSpec((1, NE), lambda i: (0, 0)),
        ],
        out_shape=[
            jax.ShapeDtypeStruct((NBLK, 1, BLK), jnp.int32),
            jax.ShapeDtypeStruct((1, NE), jnp.float32),
        ],
    )(x, vqt)


def _sc_gather_scatter(x3, vq, idx2, zeros):
    mesh = plsc.VectorSubcoreMesh(core_axis_name="c", subcore_axis_name="s")

    @functools.partial(
        pl.kernel,
        out_type=[
            jax.ShapeDtypeStruct((NW, NCH, CHUNK, EMB), jnp.float32),
            jax.ShapeDtypeStruct((NC, NE, EMB), jnp.float32),
        ],
        mesh=mesh,
        compiler_params=pltpu.CompilerParams(use_tc_tiling_on_sc=False),
        scratch_types=[
            pltpu.VMEM((NCH, CHUNK), jnp.int32),
            pltpu.VMEM((NCH, CHUNK, EMB), jnp.float32),
            pltpu.VMEM_SHARED((NE, EMB), jnp.float32),
            pltpu.SemaphoreType.DMA,
        ],
    )
    def body(x_hbm, vq_hbm, idx_hbm, z_hbm, quant_hbm, csum_hbm,
             idx_v, buf_v, shared, sem):
        c = lax.axis_index("c")
        s = lax.axis_index("s")
        wid = s * NC + c
        pltpu.sync_copy(z_hbm.at[pl.ds(s * SC_ROWS, SC_ROWS)],
                        shared.at[pl.ds(s * SC_ROWS, SC_ROWS)])
        pltpu.sync_copy(idx_hbm.at[wid], idx_v)
        copies = [
            pltpu.async_copy(vq_hbm.at[idx_v.at[j]], buf_v.at[j], sem)
            for j in range(NCH)
        ]
        for cp in copies:
            cp.wait()
        pltpu.sync_copy(buf_v, quant_hbm.at[wid])
        pltpu.sync_copy(x_hbm.at[wid], buf_v)
        plsc.subcore_barrier()
        for j in range(NCH):
            pltpu.sync_copy(buf_v.at[j], shared.at[idx_v.at[j]], add=True)
        plsc.subcore_barrier()
        pltpu.sync_copy(shared.at[pl.ds(s * SC_ROWS, SC_ROWS)],
                        csum_hbm.at[c, pl.ds(s * SC_ROWS, SC_ROWS)])

    return body(x3, vq, idx2, zeros)


def _ewma_body(cs_ref, cnt_ref, es_ref, en_ref, ns_ref, nn_ref, nvq_ref):
    cs = cs_ref[0] + cs_ref[1]
    new_sum = es_ref[...] * GAMMA + cs * (1.0 - GAMMA)
    new_n = en_ref[...] * GAMMA + cnt_ref[...] * (1.0 - GAMMA)
    ns_ref[...] = new_sum
    nn_ref[...] = new_n
    nvq_ref[...] = new_sum / new_n


def _ewma(csum2, counts, ewma_sum, ewma_n):
    return pl.pallas_call(
        _ewma_body,
        out_shape=[
            jax.ShapeDtypeStruct((NE, EMB), jnp.float32),
            jax.ShapeDtypeStruct((NE, 1), jnp.float32),
            jax.ShapeDtypeStruct((NE, EMB), jnp.float32),
        ],
    )(csum2, counts, ewma_sum, ewma_n)


def kernel(x, vq, ewma_centroid_sum, ewma_centroid_n):
    vqt = vq.T
    idx3, counts = _dist_argmin(x, vqt)
    idx2 = idx3.reshape(NW, NCH, CHUNK)
    x3 = x.reshape(NW, NCH, CHUNK, EMB)
    zeros = jnp.zeros((NE, EMB), jnp.float32)
    quant3, csum2 = _sc_gather_scatter(x3, vq, idx2, zeros)
    new_sum, new_n, new_vq = _ewma(
        csum2, counts.reshape(NE, 1), ewma_centroid_sum,
        ewma_centroid_n.reshape(NE, 1))
    return (quant3.reshape(NT, EMB), new_vq, new_sum, new_n.reshape(NE))


# speedup vs baseline: 1.1847x; 1.1847x over previous
import functools

import jax
import jax.numpy as jnp
from jax import lax
from jax.experimental import pallas as pl
from jax.experimental.pallas import tpu as pltpu
from jax.experimental.pallas import tpu_sc as plsc

EMB = 64
NE = 1024
NT = 36864
GAMMA = 0.99

NC = 2
NS = 16
NW = NC * NS
ROWS_PER_W = NT // NW
CHUNK = 128
NCH = ROWS_PER_W // CHUNK
SC_ROWS = NE // NS

BLK = ROWS_PER_W
NBLK = NT // BLK


def _dist_argmin_body(x_ref, vq_ref, idx_ref, quant_ref, counts_ref):
    i = pl.program_id(0)
    xb = x_ref[...]
    vb = vq_ref[...]
    dot = lax.dot_general(xb, vb, (((1,), (1,)), ((), ())),
                          preferred_element_type=jnp.float32)
    xs = jnp.sum(xb * xb, axis=1, keepdims=True)
    vs = jnp.sum(vb * vb, axis=1, keepdims=True).reshape(1, NE)
    d = xs - 2.0 * dot + vs
    m = jnp.min(d, axis=1, keepdims=True)
    iota = lax.broadcasted_iota(jnp.int32, d.shape, 1)
    idx = jnp.min(jnp.where(d <= m, iota, jnp.int32(NE)), axis=1)
    idx_ref[0] = idx.reshape(NCH, CHUNK)
    onehot = (iota == idx[:, None]).astype(jnp.float32)
    quant_ref[...] = lax.dot_general(onehot, vb, (((1,), (0,)), ((), ())),
                                     preferred_element_type=jnp.float32)
    ones = jnp.ones((1, BLK), jnp.float32)
    partial = lax.dot_general(ones, onehot, (((1,), (0,)), ((), ())),
                              preferred_element_type=jnp.float32)

    @pl.when(i == 0)
    def _():
        counts_ref[...] = jnp.zeros_like(counts_ref)

    counts_ref[...] += partial


def _dist_argmin(x, vq):
    return pl.pallas_call(
        _dist_argmin_body,
        grid=(NBLK,),
        in_specs=[
            pl.BlockSpec((BLK, EMB), lambda i: (i, 0)),
            pl.BlockSpec((NE, EMB), lambda i: (0, 0)),
        ],
        out_specs=[
            pl.BlockSpec((1, NCH, CHUNK), lambda i: (i, 0, 0)),
            pl.BlockSpec((BLK, EMB), lambda i: (i, 0)),
            pl.BlockSpec((1, NE), lambda i: (0, 0)),
        ],
        out_shape=[
            jax.ShapeDtypeStruct((NW, NCH, CHUNK), jnp.int32),
            jax.ShapeDtypeStruct((NT, EMB), jnp.float32),
            jax.ShapeDtypeStruct((1, NE), jnp.float32),
        ],
    )(x, vq)


def _sc_scatter(x, idx2, zeros):
    mesh = plsc.VectorSubcoreMesh(core_axis_name="c", subcore_axis_name="s")

    @functools.partial(
        pl.kernel,
        out_type=jax.ShapeDtypeStruct((NC, NE, EMB), jnp.float32),
        mesh=mesh,
        compiler_params=pltpu.CompilerParams(use_tc_tiling_on_sc=False),
        scratch_types=[
            pltpu.VMEM((NCH, CHUNK), jnp.int32),
            pltpu.VMEM((ROWS_PER_W, EMB), jnp.float32),
            pltpu.VMEM_SHARED((NE, EMB), jnp.float32),
        ],
    )
    def body(x_hbm, idx_hbm, z_hbm, csum_hbm, idx_v, buf_v, shared):
        c = lax.axis_index("c")
        s = lax.axis_index("s")
        wid = s * NC + c
        pltpu.sync_copy(z_hbm.at[pl.ds(s * SC_ROWS, SC_ROWS)],
                        shared.at[pl.ds(s * SC_ROWS, SC_ROWS)])
        pltpu.sync_copy(idx_hbm.at[wid], idx_v)
        pltpu.sync_copy(x_hbm.at[pl.ds(wid * ROWS_PER_W, ROWS_PER_W)], buf_v)
        plsc.subcore_barrier()
        for j in range(NCH):
            pltpu.sync_copy(buf_v.at[pl.ds(j * CHUNK, CHUNK)],
                            shared.at[idx_v.at[j]], add=True)
        plsc.subcore_barrier()
        pltpu.sync_copy(shared.at[pl.ds(s * SC_ROWS, SC_ROWS)],
                        csum_hbm.at[c, pl.ds(s * SC_ROWS, SC_ROWS)])

    return body(x, idx2, zeros)


def _ewma_body(cs_ref, cnt_ref, es_ref, en_ref, ns_ref, nn_ref, nvq_ref):
    cs = cs_ref[0] + cs_ref[1]
    new_sum = es_ref[...] * GAMMA + cs * (1.0 - GAMMA)
    new_n = en_ref[...] * GAMMA + cnt_ref[...] * (1.0 - GAMMA)
    ns_ref[...] = new_sum
    nn_ref[...] = new_n
    nvq_ref[...] = new_sum / new_n


def _ewma(csum2, counts, ewma_sum, ewma_n):
    return pl.pallas_call(
        _ewma_body,
        out_shape=[
            jax.ShapeDtypeStruct((NE, EMB), jnp.float32),
            jax.ShapeDtypeStruct((NE, 1), jnp.float32),
            jax.ShapeDtypeStruct((NE, EMB), jnp.float32),
        ],
    )(csum2, counts, ewma_sum, ewma_n)


def kernel(x, vq, ewma_centroid_sum, ewma_centroid_n):
    idx2, quant, counts = _dist_argmin(x, vq)
    zeros = jnp.zeros((NE, EMB), jnp.float32)
    csum2 = _sc_scatter(x, idx2, zeros)
    new_sum, new_n, new_vq = _ewma(
        csum2, counts.reshape(NE, 1), ewma_centroid_sum,
        ewma_centroid_n.reshape(NE, 1))
    return (quant, new_vq, new_sum, new_n.reshape(NE))


# speedup vs baseline: 1.2989x; 1.0964x over previous
import functools

import jax
import jax.numpy as jnp
from jax import lax
from jax.experimental import pallas as pl
from jax.experimental.pallas import tpu as pltpu
from jax.experimental.pallas import tpu_sc as plsc

EMB = 64
NE = 1024
NT = 36864
GAMMA = 0.99

NC = 2
NS = 16
NW = NC * NS
ROWS_PER_W = NT // NW
CHUNK = 128
NCH = ROWS_PER_W // CHUNK
SC_ROWS = NE // NS

BLK = ROWS_PER_W
NBLK = NT // BLK


def _dist_argmin_body(xt_ref, vq_ref, vqt_ref, idx_ref, quant_ref, counts_ref):
    i = pl.program_id(0)
    xb = xt_ref[...]
    vqb = vq_ref[...]
    vtb = vqt_ref[...]
    dot = lax.dot_general(vqb, xb, (((1,), (0,)), ((), ())),
                          preferred_element_type=jnp.float32)
    xs = jnp.sum(xb * xb, axis=0, keepdims=True)
    vs = jnp.sum(vqb * vqb, axis=1, keepdims=True)
    d = xs - 2.0 * dot + vs
    m = jnp.min(d, axis=0, keepdims=True)
    iota = lax.broadcasted_iota(jnp.int32, d.shape, 0)
    idx = jnp.min(jnp.where(d <= m, iota, jnp.int32(NE)), axis=0)
    idx_ref[0, 0] = idx
    onehot = (iota == idx[None, :]).astype(jnp.float32)
    quant_ref[...] = lax.dot_general(vtb, onehot, (((1,), (0,)), ((), ())),
                                     preferred_element_type=jnp.float32)
    ones = jnp.ones((BLK, 1), jnp.float32)
    partial = lax.dot_general(onehot, ones, (((1,), (0,)), ((), ())),
                              preferred_element_type=jnp.float32)

    @pl.when(i == 0)
    def _():
        counts_ref[...] = jnp.zeros_like(counts_ref)

    counts_ref[...] += partial


def _dist_argmin(xt, vq, vqt):
    return pl.pallas_call(
        _dist_argmin_body,
        grid=(NBLK,),
        in_specs=[
            pl.BlockSpec((EMB, BLK), lambda i: (0, i)),
            pl.BlockSpec((NE, EMB), lambda i: (0, 0)),
            pl.BlockSpec((EMB, NE), lambda i: (0, 0)),
        ],
        out_specs=[
            pl.BlockSpec((1, 1, BLK), lambda i: (i, 0, 0)),
            pl.BlockSpec((EMB, BLK), lambda i: (0, i)),
            pl.BlockSpec((NE, 1), lambda i: (0, 0)),
        ],
        out_shape=[
            jax.ShapeDtypeStruct((NBLK, 1, BLK), jnp.int32),
            jax.ShapeDtypeStruct((EMB, NT), jnp.float32),
            jax.ShapeDtypeStruct((NE, 1), jnp.float32),
        ],
    )(xt, vq, vqt)


def _sc_scatter(x, idx2, zeros):
    mesh = plsc.VectorSubcoreMesh(core_axis_name="c", subcore_axis_name="s")

    @functools.partial(
        pl.kernel,
        out_type=jax.ShapeDtypeStruct((NC, NE, EMB), jnp.float32),
        mesh=mesh,
        compiler_params=pltpu.CompilerParams(use_tc_tiling_on_sc=False),
        scratch_types=[
            pltpu.VMEM((NCH, CHUNK), jnp.int32),
            pltpu.VMEM((ROWS_PER_W, EMB), jnp.float32),
            pltpu.VMEM_SHARED((NE, EMB), jnp.float32),
        ],
    )
    def body(x_hbm, idx_hbm, z_hbm, csum_hbm, idx_v, buf_v, shared):
        c = lax.axis_index("c")
        s = lax.axis_index("s")
        wid = s * NC + c
        pltpu.sync_copy(z_hbm.at[pl.ds(s * SC_ROWS, SC_ROWS)],
                        shared.at[pl.ds(s * SC_ROWS, SC_ROWS)])
        pltpu.sync_copy(idx_hbm.at[wid], idx_v)
        pltpu.sync_copy(x_hbm.at[pl.ds(wid * ROWS_PER_W, ROWS_PER_W)], buf_v)
        plsc.subcore_barrier()
        for j in range(NCH):
            pltpu.sync_copy(buf_v.at[pl.ds(j * CHUNK, CHUNK)],
                            shared.at[idx_v.at[j]], add=True)
        plsc.subcore_barrier()
        pltpu.sync_copy(shared.at[pl.ds(s * SC_ROWS, SC_ROWS)],
                        csum_hbm.at[c, pl.ds(s * SC_ROWS, SC_ROWS)])

    return body(x, idx2, zeros)


def _ewma_body(cs_ref, cnt_ref, es_ref, en_ref, ns_ref, nn_ref, nvq_ref):
    cs = cs_ref[0] + cs_ref[1]
    new_sum = es_ref[...] * GAMMA + cs * (1.0 - GAMMA)
    new_n = en_ref[...] * GAMMA + cnt_ref[...] * (1.0 - GAMMA)
    ns_ref[...] = new_sum
    nn_ref[...] = new_n
    nvq_ref[...] = new_sum / new_n


def _ewma(csum2, counts, ewma_sum, ewma_n):
    return pl.pallas_call(
        _ewma_body,
        out_shape=[
            jax.ShapeDtypeStruct((NE, EMB), jnp.float32),
            jax.ShapeDtypeStruct((NE, 1), jnp.float32),
            jax.ShapeDtypeStruct((NE, EMB), jnp.float32),
        ],
    )(csum2, counts, ewma_sum, ewma_n)


def kernel(x, vq, ewma_centroid_sum, ewma_centroid_n):
    idx3, quant_t, counts = _dist_argmin(x.T, vq, vq.T)
    idx2 = idx3.reshape(NW, NCH, CHUNK)
    zeros = jnp.zeros((NE, EMB), jnp.float32)
    csum2 = _sc_scatter(x, idx2, zeros)
    new_sum, new_n, new_vq = _ewma(
        csum2, counts, ewma_centroid_sum, ewma_centroid_n.reshape(NE, 1))
    return (quant_t.T, new_vq, new_sum, new_n.reshape(NE))


# speedup vs baseline: 1.3283x; 1.0226x over previous
import functools

import jax
import jax.numpy as jnp
from jax import lax
from jax.experimental import pallas as pl
from jax.experimental.pallas import tpu as pltpu
from jax.experimental.pallas import tpu_sc as plsc

EMB = 64
NE = 1024
NT = 36864
GAMMA = 0.99

NC = 2
NS = 16
NW = NC * NS
ROWS_PER_W = NT // NW
CHUNK = 128
NCH = ROWS_PER_W // CHUNK
SC_ROWS = NE // NS

BLK = ROWS_PER_W
NBLK = NT // BLK


def _dist_argmin_body(xt_ref, vq_ref, vqt_ref, idx_ref, quant_ref, counts_ref):
    i = pl.program_id(0)
    xb = xt_ref[...]
    vqb = vq_ref[...]
    vtb = vqt_ref[...]
    dot = lax.dot_general(vqb, xb, (((1,), (0,)), ((), ())),
                          preferred_element_type=jnp.float32)
    xs = jnp.sum(xb * xb, axis=0, keepdims=True)
    vs = jnp.sum(vqb * vqb, axis=1, keepdims=True)
    d = xs - 2.0 * dot + vs
    m = jnp.min(d, axis=0, keepdims=True)
    iota = lax.broadcasted_iota(jnp.int32, d.shape, 0)
    idx = jnp.min(jnp.where(d <= m, iota, jnp.int32(NE)), axis=0)
    idx_ref[0, 0] = idx
    onehot = (iota == idx[None, :]).astype(jnp.float32)
    quant_ref[...] = lax.dot_general(vtb, onehot, (((1,), (0,)), ((), ())),
                                     preferred_element_type=jnp.float32)
    ones = jnp.ones((BLK, 1), jnp.float32)
    partial = lax.dot_general(onehot, ones, (((1,), (0,)), ((), ())),
                              preferred_element_type=jnp.float32)

    @pl.when(i == 0)
    def _():
        counts_ref[...] = jnp.zeros_like(counts_ref)

    counts_ref[...] += partial


def _dist_argmin(xt, vq, vqt):
    return pl.pallas_call(
        _dist_argmin_body,
        grid=(NBLK,),
        in_specs=[
            pl.BlockSpec((EMB, BLK), lambda i: (0, i)),
            pl.BlockSpec((NE, EMB), lambda i: (0, 0)),
            pl.BlockSpec((EMB, NE), lambda i: (0, 0)),
        ],
        out_specs=[
            pl.BlockSpec((1, 1, BLK), lambda i: (i, 0, 0)),
            pl.BlockSpec((EMB, BLK), lambda i: (0, i)),
            pl.BlockSpec((NE, 1), lambda i: (0, 0)),
        ],
        out_shape=[
            jax.ShapeDtypeStruct((NBLK, 1, BLK), jnp.int32),
            jax.ShapeDtypeStruct((EMB, NT), jnp.float32),
            jax.ShapeDtypeStruct((NE, 1), jnp.float32),
        ],
    )(xt, vq, vqt)


def _sc_transpose(xt):
    mesh = plsc.VectorSubcoreMesh(core_axis_name="c", subcore_axis_name="s")

    @functools.partial(
        pl.kernel,
        out_type=jax.ShapeDtypeStruct((NT * EMB,), jnp.float32),
        mesh=mesh,
        compiler_params=pltpu.CompilerParams(use_tc_tiling_on_sc=False,
                                             needs_layout_passes=False),
        scratch_types=[
            pltpu.VMEM((EMB, CHUNK), jnp.float32),
            pltpu.VMEM((CHUNK * EMB,), jnp.float32),
        ],
    )
    def body(xt_hbm, xf_hbm, in_v, out_v):
        c = lax.axis_index("c")
        s = lax.axis_index("s")
        wid = s * NC + c
        base_t = wid * ROWS_PER_W
        lane = jax.lax.iota(jnp.int32, 16)
        lane64 = lane * EMB
        for j in range(NCH):
            pltpu.sync_copy(xt_hbm.at[:, pl.ds(base_t + j * CHUNK, CHUNK)],
                            in_v)

            def body_d(d, carry):
                dvec = jnp.full((16,), d, jnp.int32)
                for t0 in range(CHUNK // 16):
                    vals = plsc.load_gather(in_v, [dvec, t0 * 16 + lane])
                    plsc.store_scatter(out_v, [lane64 + (t0 * 16 * EMB + d)],
                                       vals)
                return carry

            lax.fori_loop(0, EMB, body_d, 0)
            pltpu.sync_copy(
                out_v,
                xf_hbm.at[pl.ds((base_t + j * CHUNK) * EMB, CHUNK * EMB)])

    return body(xt)


def _sc_scatter(x, idx2, zeros):
    mesh = plsc.VectorSubcoreMesh(core_axis_name="c", subcore_axis_name="s")

    @functools.partial(
        pl.kernel,
        out_type=jax.ShapeDtypeStruct((NC, NE, EMB), jnp.float32),
        mesh=mesh,
        compiler_params=pltpu.CompilerParams(use_tc_tiling_on_sc=False),
        scratch_types=[
            pltpu.VMEM((NCH, CHUNK), jnp.int32),
            pltpu.VMEM((ROWS_PER_W, EMB), jnp.float32),
            pltpu.VMEM_SHARED((NE, EMB), jnp.float32),
        ],
    )
    def body(x_hbm, idx_hbm, z_hbm, csum_hbm, idx_v, buf_v, shared):
        c = lax.axis_index("c")
        s = lax.axis_index("s")
        wid = s * NC + c
        pltpu.sync_copy(z_hbm.at[pl.ds(s * SC_ROWS, SC_ROWS)],
                        shared.at[pl.ds(s * SC_ROWS, SC_ROWS)])
        pltpu.sync_copy(idx_hbm.at[wid], idx_v)
        pltpu.sync_copy(x_hbm.at[pl.ds(wid * ROWS_PER_W, ROWS_PER_W)], buf_v)
        plsc.subcore_barrier()
        for j in range(NCH):
            pltpu.sync_copy(buf_v.at[pl.ds(j * CHUNK, CHUNK)],
                            shared.at[idx_v.at[j]], add=True)
        plsc.subcore_barrier()
        pltpu.sync_copy(shared.at[pl.ds(s * SC_ROWS, SC_ROWS)],
                        csum_hbm.at[c, pl.ds(s * SC_ROWS, SC_ROWS)])

    return body(x, idx2, zeros)


def _ewma_body(cs_ref, cnt_ref, es_ref, en_ref, ns_ref, nn_ref, nvq_ref):
    cs = cs_ref[0] + cs_ref[1]
    new_sum = es_ref[...] * GAMMA + cs * (1.0 - GAMMA)
    new_n = en_ref[...] * GAMMA + cnt_ref[...] * (1.0 - GAMMA)
    ns_ref[...] = new_sum
    nn_ref[...] = new_n
    nvq_ref[...] = new_sum / new_n


def _ewma(csum2, counts, ewma_sum, ewma_n):
    return pl.pallas_call(
        _ewma_body,
        out_shape=[
            jax.ShapeDtypeStruct((NE, EMB), jnp.float32),
            jax.ShapeDtypeStruct((NE, 1), jnp.float32),
            jax.ShapeDtypeStruct((NE, EMB), jnp.float32),
        ],
    )(csum2, counts, ewma_sum, ewma_n)


def kernel(x, vq, ewma_centroid_sum, ewma_centroid_n):
    xt = x.T
    x_flat = _sc_transpose(xt)
    idx3, quant_t, counts = _dist_argmin(xt, vq, vq.T)
    idx2 = idx3.reshape(NW, NCH, CHUNK)
    zeros = jnp.zeros((NE, EMB), jnp.float32)
    csum2 = _sc_scatter(x_flat.reshape(NT, EMB), idx2, zeros)
    new_sum, new_n, new_vq = _ewma(
        csum2, counts, ewma_centroid_sum, ewma_centroid_n.reshape(NE, 1))
    return (quant_t.T, new_vq, new_sum, new_n.reshape(NE))


# speedup vs baseline: 1.5068x; 1.1344x over previous
import functools

import jax
import jax.numpy as jnp
from jax import lax
from jax.experimental import pallas as pl
from jax.experimental.pallas import tpu as pltpu
from jax.experimental.pallas import tpu_sc as plsc

EMB = 64
NE = 1024
NT = 36864
GAMMA = 0.99

NC = 2
NS = 16
NW = NC * NS
ROWS_PER_W = NT // NW
CHUNK = 128
NCH = ROWS_PER_W // CHUNK
SC_ROWS = NE // NS

BLK = ROWS_PER_W
NBLK = NT // BLK


def _dist_argmin_body(xt_ref, vq_ref, vqt_ref, idx_ref, quant_ref, counts_ref):
    i = pl.program_id(0)
    xb = xt_ref[...]
    vqb = vq_ref[...]
    vtb = vqt_ref[...]
    dot = lax.dot_general(vqb, xb, (((1,), (0,)), ((), ())),
                          preferred_element_type=jnp.float32)
    xs = jnp.sum(xb * xb, axis=0, keepdims=True)
    vs = jnp.sum(vqb * vqb, axis=1, keepdims=True)
    d = xs - 2.0 * dot + vs
    idx = jnp.argmin(d, axis=0).astype(jnp.int32)
    iota = lax.broadcasted_iota(jnp.int32, d.shape, 0)
    idx_ref[0, 0] = idx
    onehot = (iota == idx[None, :]).astype(jnp.float32)
    quant_ref[...] = lax.dot_general(vtb, onehot, (((1,), (0,)), ((), ())),
                                     preferred_element_type=jnp.float32)
    ones = jnp.ones((BLK, 1), jnp.float32)
    partial = lax.dot_general(onehot, ones, (((1,), (0,)), ((), ())),
                              preferred_element_type=jnp.float32)

    @pl.when(i == 0)
    def _():
        counts_ref[...] = jnp.zeros_like(counts_ref)

    counts_ref[...] += partial


def _dist_argmin(xt, vq, vqt):
    return pl.pallas_call(
        _dist_argmin_body,
        grid=(NBLK,),
        in_specs=[
            pl.BlockSpec((EMB, BLK), lambda i: (0, i)),
            pl.BlockSpec((NE, EMB), lambda i: (0, 0)),
            pl.BlockSpec((EMB, NE), lambda i: (0, 0)),
        ],
        out_specs=[
            pl.BlockSpec((1, 1, BLK), lambda i: (i, 0, 0)),
            pl.BlockSpec((EMB, BLK), lambda i: (0, i)),
            pl.BlockSpec((NE, 1), lambda i: (0, 0)),
        ],
        out_shape=[
            jax.ShapeDtypeStruct((NBLK, 1, BLK), jnp.int32),
            jax.ShapeDtypeStruct((EMB, NT), jnp.float32),
            jax.ShapeDtypeStruct((NE, 1), jnp.float32),
        ],
    )(xt, vq, vqt)


def _sc_transpose(xt):
    mesh = plsc.VectorSubcoreMesh(core_axis_name="c", subcore_axis_name="s")

    @functools.partial(
        pl.kernel,
        out_type=jax.ShapeDtypeStruct((NT * EMB,), jnp.float32),
        mesh=mesh,
        compiler_params=pltpu.CompilerParams(use_tc_tiling_on_sc=False,
                                             needs_layout_passes=False),
        scratch_types=[
            pltpu.VMEM((EMB, CHUNK), jnp.float32),
            pltpu.VMEM((CHUNK * EMB,), jnp.float32),
        ],
    )
    def body(xt_hbm, xf_hbm, in_v, out_v):
        c = lax.axis_index("c")
        s = lax.axis_index("s")
        wid = s * NC + c
        base_t = wid * ROWS_PER_W
        lane = jax.lax.iota(jnp.int32, 16)
        lane64 = lane * EMB
        for j in range(NCH):
            pltpu.sync_copy(xt_hbm.at[:, pl.ds(base_t + j * CHUNK, CHUNK)],
                            in_v)

            def body_d(d, carry):
                dvec = jnp.full((16,), d, jnp.int32)
                for t0 in range(CHUNK // 16):
                    vals = plsc.load_gather(in_v, [dvec, t0 * 16 + lane])
                    plsc.store_scatter(out_v, [lane64 + (t0 * 16 * EMB + d)],
                                       vals)
                return carry

            lax.fori_loop(0, EMB, body_d, 0)
            pltpu.sync_copy(
                out_v,
                xf_hbm.at[pl.ds((base_t + j * CHUNK) * EMB, CHUNK * EMB)])

    return body(xt)


def _sc_scatter(x, idx2, zeros):
    mesh = plsc.VectorSubcoreMesh(core_axis_name="c", subcore_axis_name="s")

    @functools.partial(
        pl.kernel,
        out_type=jax.ShapeDtypeStruct((NC, NE, EMB), jnp.float32),
        mesh=mesh,
        compiler_params=pltpu.CompilerParams(use_tc_tiling_on_sc=False),
        scratch_types=[
            pltpu.VMEM((NCH, CHUNK), jnp.int32),
            pltpu.VMEM((ROWS_PER_W, EMB), jnp.float32),
            pltpu.VMEM_SHARED((NE, EMB), jnp.float32),
        ],
    )
    def body(x_hbm, idx_hbm, z_hbm, csum_hbm, idx_v, buf_v, shared):
        c = lax.axis_index("c")
        s = lax.axis_index("s")
        wid = s * NC + c
        pltpu.sync_copy(z_hbm.at[pl.ds(s * SC_ROWS, SC_ROWS)],
                        shared.at[pl.ds(s * SC_ROWS, SC_ROWS)])
        pltpu.sync_copy(idx_hbm.at[wid], idx_v)
        pltpu.sync_copy(x_hbm.at[pl.ds(wid * ROWS_PER_W, ROWS_PER_W)], buf_v)
        plsc.subcore_barrier()
        for j in range(NCH):
            pltpu.sync_copy(buf_v.at[pl.ds(j * CHUNK, CHUNK)],
                            shared.at[idx_v.at[j]], add=True)
        plsc.subcore_barrier()
        pltpu.sync_copy(shared.at[pl.ds(s * SC_ROWS, SC_ROWS)],
                        csum_hbm.at[c, pl.ds(s * SC_ROWS, SC_ROWS)])

    return body(x, idx2, zeros)


def _ewma_body(cs_ref, cnt_ref, es_ref, en_ref, ns_ref, nn_ref, nvq_ref):
    cs = cs_ref[0] + cs_ref[1]
    new_sum = es_ref[...] * GAMMA + cs * (1.0 - GAMMA)
    new_n = en_ref[...] * GAMMA + cnt_ref[...] * (1.0 - GAMMA)
    ns_ref[...] = new_sum
    nn_ref[...] = new_n
    nvq_ref[...] = new_sum / new_n


def _ewma(csum2, counts, ewma_sum, ewma_n):
    return pl.pallas_call(
        _ewma_body,
        out_shape=[
            jax.ShapeDtypeStruct((NE, EMB), jnp.float32),
            jax.ShapeDtypeStruct((NE, 1), jnp.float32),
            jax.ShapeDtypeStruct((NE, EMB), jnp.float32),
        ],
    )(csum2, counts, ewma_sum, ewma_n)


def kernel(x, vq, ewma_centroid_sum, ewma_centroid_n):
    xt = x.T
    x_flat = _sc_transpose(xt)
    idx3, quant_t, counts = _dist_argmin(xt, vq, vq.T)
    idx2 = idx3.reshape(NW, NCH, CHUNK)
    zeros = jnp.zeros((NE, EMB), jnp.float32)
    csum2 = _sc_scatter(x_flat.reshape(NT, EMB), idx2, zeros)
    new_sum, new_n, new_vq = _ewma(
        csum2, counts, ewma_centroid_sum, ewma_centroid_n.reshape(NE, 1))
    return (quant_t.T, new_vq, new_sum, new_n.reshape(NE))


# speedup vs baseline: 1.7744x; 1.1776x over previous
import functools

import jax
import jax.numpy as jnp
from jax import lax
from jax.experimental import pallas as pl
from jax.experimental.pallas import tpu as pltpu
from jax.experimental.pallas import tpu_sc as plsc

EMB = 64
NE = 1024
NT = 36864
GAMMA = 0.99

NC = 2
NS = 16
NW = NC * NS
ROWS_PER_W = NT // NW
CHUNK = 128
NCH = ROWS_PER_W // CHUNK
SC_ROWS = NE // NS

BLK = ROWS_PER_W
NBLK = NT // BLK


def _dist_argmin_body(xt_ref, vq_ref, vqt_ref, idx_ref, quant_ref):
    xb = xt_ref[...]
    vqb = vq_ref[...]
    vtb = vqt_ref[...]
    dot = lax.dot_general(vqb, xb, (((1,), (0,)), ((), ())),
                          preferred_element_type=jnp.float32)
    xs = jnp.sum(xb * xb, axis=0, keepdims=True)
    vs = jnp.sum(vqb * vqb, axis=1, keepdims=True)
    d = xs - 2.0 * dot + vs
    idx = jnp.argmin(d, axis=0).astype(jnp.int32)
    iota = lax.broadcasted_iota(jnp.int32, d.shape, 0)
    idx_ref[0, 0] = idx
    onehot = (iota == idx[None, :]).astype(jnp.float32)
    quant_ref[...] = lax.dot_general(vtb, onehot, (((1,), (0,)), ((), ())),
                                     preferred_element_type=jnp.float32)


def _dist_argmin(xt, vq, vqt):
    return pl.pallas_call(
        _dist_argmin_body,
        grid=(NBLK,),
        in_specs=[
            pl.BlockSpec((EMB, BLK), lambda i: (0, i)),
            pl.BlockSpec((NE, EMB), lambda i: (0, 0)),
            pl.BlockSpec((EMB, NE), lambda i: (0, 0)),
        ],
        out_specs=[
            pl.BlockSpec((1, 1, BLK), lambda i: (i, 0, 0)),
            pl.BlockSpec((EMB, BLK), lambda i: (0, i)),
        ],
        out_shape=[
            jax.ShapeDtypeStruct((NBLK, 1, BLK), jnp.int32),
            jax.ShapeDtypeStruct((EMB, NT), jnp.float32),
        ],
    )(xt, vq, vqt)


def _sc_transpose(xt):
    mesh = plsc.VectorSubcoreMesh(core_axis_name="c", subcore_axis_name="s")

    @functools.partial(
        pl.kernel,
        out_type=jax.ShapeDtypeStruct((NT * EMB,), jnp.float32),
        mesh=mesh,
        compiler_params=pltpu.CompilerParams(use_tc_tiling_on_sc=False,
                                             needs_layout_passes=False),
        scratch_types=[
            pltpu.VMEM((EMB, CHUNK), jnp.float32),
            pltpu.VMEM((CHUNK * EMB,), jnp.float32),
        ],
    )
    def body(xt_hbm, xf_hbm, in_v, out_v):
        c = lax.axis_index("c")
        s = lax.axis_index("s")
        wid = s * NC + c
        base_t = wid * ROWS_PER_W
        lane = jax.lax.iota(jnp.int32, 16)
        lane64 = lane * EMB
        for j in range(NCH):
            pltpu.sync_copy(xt_hbm.at[:, pl.ds(base_t + j * CHUNK, CHUNK)],
                            in_v)

            def body_d(d, carry):
                dvec = jnp.full((16,), d, jnp.int32)
                for t0 in range(CHUNK // 16):
                    vals = plsc.load_gather(in_v, [dvec, t0 * 16 + lane])
                    plsc.store_scatter(out_v, [lane64 + (t0 * 16 * EMB + d)],
                                       vals)
                return carry

            lax.fori_loop(0, EMB, body_d, 0)
            pltpu.sync_copy(
                out_v,
                xf_hbm.at[pl.ds((base_t + j * CHUNK) * EMB, CHUNK * EMB)])

    return body(xt)


def _sc_scatter(x, idx2, zeros, zeros_n, ones):
    mesh = plsc.VectorSubcoreMesh(core_axis_name="c", subcore_axis_name="s")

    @functools.partial(
        pl.kernel,
        out_type=[
            jax.ShapeDtypeStruct((NC, NE, EMB), jnp.float32),
            jax.ShapeDtypeStruct((NC, NE, 16), jnp.float32),
        ],
        mesh=mesh,
        compiler_params=pltpu.CompilerParams(use_tc_tiling_on_sc=False),
        scratch_types=[
            pltpu.VMEM((NCH, CHUNK), jnp.int32),
            pltpu.VMEM((ROWS_PER_W, EMB), jnp.float32),
            pltpu.VMEM((CHUNK, 16), jnp.float32),
            pltpu.VMEM_SHARED((NE, EMB), jnp.float32),
            pltpu.VMEM_SHARED((NE, 16), jnp.float32),
        ],
    )
    def body(x_hbm, idx_hbm, z_hbm, zn_hbm, ones_hbm, csum_hbm, cnt_hbm,
             idx_v, buf_v, ones_v, shared, shared_n):
        c = lax.axis_index("c")
        s = lax.axis_index("s")
        wid = s * NC + c
        pltpu.sync_copy(z_hbm.at[pl.ds(s * SC_ROWS, SC_ROWS)],
                        shared.at[pl.ds(s * SC_ROWS, SC_ROWS)])
        pltpu.sync_copy(zn_hbm.at[pl.ds(s * SC_ROWS, SC_ROWS)],
                        shared_n.at[pl.ds(s * SC_ROWS, SC_ROWS)])
        pltpu.sync_copy(idx_hbm.at[wid], idx_v)
        pltpu.sync_copy(ones_hbm, ones_v)
        pltpu.sync_copy(x_hbm.at[pl.ds(wid * ROWS_PER_W, ROWS_PER_W)], buf_v)
        plsc.subcore_barrier()
        for j in range(NCH):
            pltpu.sync_copy(buf_v.at[pl.ds(j * CHUNK, CHUNK)],
                            shared.at[idx_v.at[j]], add=True)
            pltpu.sync_copy(ones_v, shared_n.at[idx_v.at[j]], add=True)
        plsc.subcore_barrier()
        pltpu.sync_copy(shared.at[pl.ds(s * SC_ROWS, SC_ROWS)],
                        csum_hbm.at[c, pl.ds(s * SC_ROWS, SC_ROWS)])
        pltpu.sync_copy(shared_n.at[pl.ds(s * SC_ROWS, SC_ROWS)],
                        cnt_hbm.at[c, pl.ds(s * SC_ROWS, SC_ROWS)])

    return body(x, idx2, zeros, zeros_n, ones)


def _ewma_body(cs_ref, cnt_ref, es_ref, en_ref, ns_ref, nn_ref, nvq_ref):
    cs = cs_ref[0] + cs_ref[1]
    cnt = cnt_ref[0, :, 0:1] + cnt_ref[1, :, 0:1]
    new_sum = es_ref[...] * GAMMA + cs * (1.0 - GAMMA)
    new_n = en_ref[...] * GAMMA + cnt * (1.0 - GAMMA)
    ns_ref[...] = new_sum
    nn_ref[...] = new_n
    nvq_ref[...] = new_sum / new_n


def _ewma(csum2, cnt2, ewma_sum, ewma_n):
    return pl.pallas_call(
        _ewma_body,
        out_shape=[
            jax.ShapeDtypeStruct((NE, EMB), jnp.float32),
            jax.ShapeDtypeStruct((NE, 1), jnp.float32),
            jax.ShapeDtypeStruct((NE, EMB), jnp.float32),
        ],
    )(csum2, cnt2, ewma_sum, ewma_n)


def kernel(x, vq, ewma_centroid_sum, ewma_centroid_n):
    xt = x.T
    x_flat = _sc_transpose(xt)
    idx3, quant_t = _dist_argmin(xt, vq, vq.T)
    idx2 = idx3.reshape(NW, NCH, CHUNK)
    zeros = jnp.zeros((NE, EMB), jnp.float32)
    zeros_n = jnp.zeros((NE, 16), jnp.float32)
    ones = jnp.ones((CHUNK, 16), jnp.float32)
    csum2, cnt2 = _sc_scatter(x_flat.reshape(NT, EMB), idx2, zeros,
                              zeros_n, ones)
    new_sum, new_n, new_vq = _ewma(
        csum2, cnt2, ewma_centroid_sum, ewma_centroid_n.reshape(NE, 1))
    return (quant_t.T, new_vq, new_sum, new_n.reshape(NE))


# speedup vs baseline: 1.8818x; 1.0605x over previous
import functools

import jax
import jax.numpy as jnp
from jax import lax
from jax.experimental import pallas as pl
from jax.experimental.pallas import tpu as pltpu
from jax.experimental.pallas import tpu_sc as plsc

EMB = 64
NE = 1024
NT = 36864
GAMMA = 0.99

NC = 2
NS = 16
NW = NC * NS
ROWS_PER_W = NT // NW
CHUNK = 128
NCH = ROWS_PER_W // CHUNK
SC_ROWS = NE // NS

BLK = ROWS_PER_W
NBLK = NT // BLK


def _dist_argmin_body(xt_ref, vq_ref, vqt_ref, idx_ref, quant_ref):
    xb = xt_ref[...]
    vqb = vq_ref[...]
    vtb = vqt_ref[...]
    dot = lax.dot_general(vqb, xb, (((1,), (0,)), ((), ())),
                          preferred_element_type=jnp.float32)
    xs = jnp.sum(xb * xb, axis=0, keepdims=True)
    vs = jnp.sum(vqb * vqb, axis=1, keepdims=True)
    d = xs - 2.0 * dot + vs
    idx = jnp.argmin(d, axis=0).astype(jnp.int32)
    iota = lax.broadcasted_iota(jnp.int32, d.shape, 0)
    idx_ref[0, 0] = idx
    onehot = (iota == idx[None, :]).astype(jnp.float32)
    quant_ref[...] = lax.dot_general(vtb, onehot, (((1,), (0,)), ((), ())),
                                     preferred_element_type=jnp.float32)


def _dist_argmin(xt, vq, vqt):
    return pl.pallas_call(
        _dist_argmin_body,
        grid=(NBLK,),
        in_specs=[
            pl.BlockSpec((EMB, BLK), lambda i: (0, i)),
            pl.BlockSpec((NE, EMB), lambda i: (0, 0)),
            pl.BlockSpec((EMB, NE), lambda i: (0, 0)),
        ],
        out_specs=[
            pl.BlockSpec((1, 1, BLK), lambda i: (i, 0, 0)),
            pl.BlockSpec((EMB, BLK), lambda i: (0, i)),
        ],
        out_shape=[
            jax.ShapeDtypeStruct((NBLK, 1, BLK), jnp.int32),
            jax.ShapeDtypeStruct((EMB, NT), jnp.float32),
        ],
    )(xt, vq, vqt)


def _sc_transpose(xt):
    mesh = plsc.VectorSubcoreMesh(core_axis_name="c", subcore_axis_name="s")

    @functools.partial(
        pl.kernel,
        out_type=jax.ShapeDtypeStruct((NT * EMB,), jnp.float32),
        mesh=mesh,
        compiler_params=pltpu.CompilerParams(use_tc_tiling_on_sc=False,
                                             needs_layout_passes=False),
        scratch_types=[
            pltpu.VMEM((EMB, ROWS_PER_W), jnp.float32),
            pltpu.VMEM((ROWS_PER_W // 2 * EMB,), jnp.float32),
        ],
    )
    def body(xt_hbm, xf_hbm, in_v, out_v):
        c = lax.axis_index("c")
        s = lax.axis_index("s")
        wid = s * NC + c
        base_t = wid * ROWS_PER_W
        half = ROWS_PER_W // 2
        lane = jax.lax.iota(jnp.int32, 16)
        lane64 = lane * EMB
        pltpu.sync_copy(xt_hbm.at[:, pl.ds(base_t, ROWS_PER_W)], in_v)
        for h in range(2):

            def body_d(d, carry):
                dvec = jnp.full((16,), d, jnp.int32)
                lane64d = lane64 + d
                for t0 in range(half // 16):
                    vals = plsc.load_gather(
                        in_v, [dvec, h * half + t0 * 16 + lane])
                    plsc.store_scatter(out_v, [lane64d + t0 * 16 * EMB], vals)
                return carry

            lax.fori_loop(0, EMB, body_d, 0)
            pltpu.sync_copy(
                out_v,
                xf_hbm.at[pl.ds((base_t + h * half) * EMB, half * EMB)])

    return body(xt)


def _sc_scatter(x, idx2, zeros, zeros_n, ones):
    mesh = plsc.VectorSubcoreMesh(core_axis_name="c", subcore_axis_name="s")

    @functools.partial(
        pl.kernel,
        out_type=[
            jax.ShapeDtypeStruct((NC, NE, EMB), jnp.float32),
            jax.ShapeDtypeStruct((NC, NE, 16), jnp.float32),
        ],
        mesh=mesh,
        compiler_params=pltpu.CompilerParams(use_tc_tiling_on_sc=False),
        scratch_types=[
            pltpu.VMEM((NCH, CHUNK), jnp.int32),
            pltpu.VMEM((ROWS_PER_W, EMB), jnp.float32),
            pltpu.VMEM((CHUNK, 16), jnp.float32),
            pltpu.VMEM_SHARED((NE, EMB), jnp.float32),
            pltpu.VMEM_SHARED((NE, 16), jnp.float32),
        ],
    )
    def body(x_hbm, idx_hbm, z_hbm, zn_hbm, ones_hbm, csum_hbm, cnt_hbm,
             idx_v, buf_v, ones_v, shared, shared_n):
        c = lax.axis_index("c")
        s = lax.axis_index("s")
        wid = s * NC + c
        pltpu.sync_copy(z_hbm.at[pl.ds(s * SC_ROWS, SC_ROWS)],
                        shared.at[pl.ds(s * SC_ROWS, SC_ROWS)])
        pltpu.sync_copy(zn_hbm.at[pl.ds(s * SC_ROWS, SC_ROWS)],
                        shared_n.at[pl.ds(s * SC_ROWS, SC_ROWS)])
        pltpu.sync_copy(idx_hbm.at[wid], idx_v)
        pltpu.sync_copy(ones_hbm, ones_v)
        pltpu.sync_copy(x_hbm.at[pl.ds(wid * ROWS_PER_W, ROWS_PER_W)], buf_v)
        plsc.subcore_barrier()
        for j in range(NCH):
            pltpu.sync_copy(buf_v.at[pl.ds(j * CHUNK, CHUNK)],
                            shared.at[idx_v.at[j]], add=True)
            pltpu.sync_copy(ones_v, shared_n.at[idx_v.at[j]], add=True)
        plsc.subcore_barrier()
        pltpu.sync_copy(shared.at[pl.ds(s * SC_ROWS, SC_ROWS)],
                        csum_hbm.at[c, pl.ds(s * SC_ROWS, SC_ROWS)])
        pltpu.sync_copy(shared_n.at[pl.ds(s * SC_ROWS, SC_ROWS)],
                        cnt_hbm.at[c, pl.ds(s * SC_ROWS, SC_ROWS)])

    return body(x, idx2, zeros, zeros_n, ones)


def _ewma_body(cs_ref, cnt_ref, es_ref, en_ref, ns_ref, nn_ref, nvq_ref):
    cs = cs_ref[0] + cs_ref[1]
    cnt = cnt_ref[0, :, 0:1] + cnt_ref[1, :, 0:1]
    new_sum = es_ref[...] * GAMMA + cs * (1.0 - GAMMA)
    new_n = en_ref[...] * GAMMA + cnt * (1.0 - GAMMA)
    ns_ref[...] = new_sum
    nn_ref[...] = new_n
    nvq_ref[...] = new_sum / new_n


def _ewma(csum2, cnt2, ewma_sum, ewma_n):
    return pl.pallas_call(
        _ewma_body,
        out_shape=[
            jax.ShapeDtypeStruct((NE, EMB), jnp.float32),
            jax.ShapeDtypeStruct((NE, 1), jnp.float32),
            jax.ShapeDtypeStruct((NE, EMB), jnp.float32),
        ],
    )(csum2, cnt2, ewma_sum, ewma_n)


def kernel(x, vq, ewma_centroid_sum, ewma_centroid_n):
    xt = x.T
    x_flat = _sc_transpose(xt)
    idx3, quant_t = _dist_argmin(xt, vq, vq.T)
    idx2 = idx3.reshape(NW, NCH, CHUNK)
    zeros = jnp.zeros((NE, EMB), jnp.float32)
    zeros_n = jnp.zeros((NE, 16), jnp.float32)
    ones = jnp.ones((CHUNK, 16), jnp.float32)
    csum2, cnt2 = _sc_scatter(x_flat.reshape(NT, EMB), idx2, zeros,
                              zeros_n, ones)
    new_sum, new_n, new_vq = _ewma(
        csum2, cnt2, ewma_centroid_sum, ewma_centroid_n.reshape(NE, 1))
    return (quant_t.T, new_vq, new_sum, new_n.reshape(NE))


# speedup vs baseline: 2.1175x; 1.1252x over previous
import functools

import jax
import jax.numpy as jnp
from jax import lax
from jax.experimental import pallas as pl
from jax.experimental.pallas import tpu as pltpu
from jax.experimental.pallas import tpu_sc as plsc

EMB = 64
NE = 1024
NT = 36864
GAMMA = 0.99

NC = 2
NS = 16
NW = NC * NS
ROWS_PER_W = NT // NW
CHUNK = 128
NCH = ROWS_PER_W // CHUNK
SC_ROWS = NE // NS

BLK = ROWS_PER_W
NBLK = NT // BLK


def _dist_argmin_body(xt_ref, vq_ref, vqt_ref, idx_ref, quant_ref):
    xb = xt_ref[...]
    vqb = vq_ref[...]
    vtb = vqt_ref[...]
    dot = lax.dot_general(vqb, xb, (((1,), (0,)), ((), ())),
                          preferred_element_type=jnp.float32)
    xs = jnp.sum(xb * xb, axis=0, keepdims=True)
    vs = jnp.sum(vqb * vqb, axis=1, keepdims=True)
    d = xs - 2.0 * dot + vs
    idx = jnp.argmin(d, axis=0).astype(jnp.int32)
    iota = lax.broadcasted_iota(jnp.int32, d.shape, 0)
    idx_ref[0, 0] = idx
    onehot = (iota == idx[None, :]).astype(jnp.float32)
    quant_ref[...] = lax.dot_general(vtb, onehot, (((1,), (0,)), ((), ())),
                                     preferred_element_type=jnp.float32)


def _dist_argmin(xt, vq, vqt):
    return pl.pallas_call(
        _dist_argmin_body,
        grid=(NBLK,),
        in_specs=[
            pl.BlockSpec((EMB, BLK), lambda i: (0, i)),
            pl.BlockSpec((NE, EMB), lambda i: (0, 0)),
            pl.BlockSpec((EMB, NE), lambda i: (0, 0)),
        ],
        out_specs=[
            pl.BlockSpec((1, 1, BLK), lambda i: (i, 0, 0)),
            pl.BlockSpec((EMB, BLK), lambda i: (0, i)),
        ],
        out_shape=[
            jax.ShapeDtypeStruct((NBLK, 1, BLK), jnp.int32),
            jax.ShapeDtypeStruct((EMB, NT), jnp.float32),
        ],
    )(xt, vq, vqt)


def _sc_transpose(x4):
    mesh = plsc.VectorSubcoreMesh(core_axis_name="c", subcore_axis_name="s")
    half = ROWS_PER_W // 2

    @functools.partial(
        pl.kernel,
        out_type=jax.ShapeDtypeStruct((NT, EMB), jnp.float32),
        mesh=mesh,
        compiler_params=pltpu.CompilerParams(use_tc_tiling_on_sc=False,
                                             needs_layout_passes=False),
        scratch_types=[
            pltpu.VMEM((8, NCH, 8, CHUNK), jnp.float32),
            pltpu.VMEM((half, EMB + 1), jnp.float32),
        ],
    )
    def body(x4_hbm, xf_hbm, in_v, out_v):
        c = lax.axis_index("c")
        s = lax.axis_index("s")
        wid = s * NC + c
        base_t = wid * ROWS_PER_W
        lane = jax.lax.iota(jnp.int32, 16)
        pltpu.sync_copy(x4_hbm.at[:, pl.ds(wid * NCH, NCH)], in_v)
        for h in range(2):

            def body_d(d, carry):
                dhi = jnp.full((16,), d // 8, jnp.int32)
                dlo = jnp.full((16,), d % 8, jnp.int32)
                dv = jnp.full((16,), d, jnp.int32)
                for t0 in range(half // 16):
                    tl = h * half + t0 * 16
                    vals = plsc.load_gather(
                        in_v, [dhi, jnp.full((16,), tl // CHUNK, jnp.int32),
                               dlo, tl % CHUNK + lane])
                    plsc.store_scatter(out_v, [t0 * 16 + lane, dv], vals)
                return carry

            lax.fori_loop(0, EMB, body_d, 0)
            pltpu.sync_copy(out_v.at[:, pl.ds(0, EMB)],
                            xf_hbm.at[pl.ds(base_t + h * half, half)])

    return body(x4)


def _sc_scatter(x, idx2, zeros, zeros_n, ones):
    mesh = plsc.VectorSubcoreMesh(core_axis_name="c", subcore_axis_name="s")

    @functools.partial(
        pl.kernel,
        out_type=[
            jax.ShapeDtypeStruct((NC, NE, EMB), jnp.float32),
            jax.ShapeDtypeStruct((NC, NE, 16), jnp.float32),
        ],
        mesh=mesh,
        compiler_params=pltpu.CompilerParams(use_tc_tiling_on_sc=False),
        scratch_types=[
            pltpu.VMEM((NCH, CHUNK), jnp.int32),
            pltpu.VMEM((ROWS_PER_W, EMB), jnp.float32),
            pltpu.VMEM((CHUNK, 16), jnp.float32),
            pltpu.VMEM_SHARED((NE, EMB), jnp.float32),
            pltpu.VMEM_SHARED((NE, 16), jnp.float32),
        ],
    )
    def body(x_hbm, idx_hbm, z_hbm, zn_hbm, ones_hbm, csum_hbm, cnt_hbm,
             idx_v, buf_v, ones_v, shared, shared_n):
        c = lax.axis_index("c")
        s = lax.axis_index("s")
        wid = s * NC + c
        pltpu.sync_copy(z_hbm.at[pl.ds(s * SC_ROWS, SC_ROWS)],
                        shared.at[pl.ds(s * SC_ROWS, SC_ROWS)])
        pltpu.sync_copy(zn_hbm.at[pl.ds(s * SC_ROWS, SC_ROWS)],
                        shared_n.at[pl.ds(s * SC_ROWS, SC_ROWS)])
        pltpu.sync_copy(idx_hbm.at[wid], idx_v)
        pltpu.sync_copy(ones_hbm, ones_v)
        pltpu.sync_copy(x_hbm.at[pl.ds(wid * ROWS_PER_W, ROWS_PER_W)], buf_v)
        plsc.subcore_barrier()
        for j in range(NCH):
            pltpu.sync_copy(buf_v.at[pl.ds(j * CHUNK, CHUNK)],
                            shared.at[idx_v.at[j]], add=True)
            pltpu.sync_copy(ones_v, shared_n.at[idx_v.at[j]], add=True)
        plsc.subcore_barrier()
        pltpu.sync_copy(shared.at[pl.ds(s * SC_ROWS, SC_ROWS)],
                        csum_hbm.at[c, pl.ds(s * SC_ROWS, SC_ROWS)])
        pltpu.sync_copy(shared_n.at[pl.ds(s * SC_ROWS, SC_ROWS)],
                        cnt_hbm.at[c, pl.ds(s * SC_ROWS, SC_ROWS)])

    return body(x, idx2, zeros, zeros_n, ones)


def _ewma_body(cs_ref, cnt_ref, es_ref, en_ref, ns_ref, nn_ref, nvq_ref):
    cs = cs_ref[0] + cs_ref[1]
    cnt = cnt_ref[0, :, 0:1] + cnt_ref[1, :, 0:1]
    new_sum = es_ref[...] * GAMMA + cs * (1.0 - GAMMA)
    new_n = en_ref[...] * GAMMA + cnt * (1.0 - GAMMA)
    ns_ref[...] = new_sum
    nn_ref[...] = new_n
    nvq_ref[...] = new_sum / new_n


def _ewma(csum2, cnt2, ewma_sum, ewma_n):
    return pl.pallas_call(
        _ewma_body,
        out_shape=[
            jax.ShapeDtypeStruct((NE, EMB), jnp.float32),
            jax.ShapeDtypeStruct((NE, 1), jnp.float32),
            jax.ShapeDtypeStruct((NE, EMB), jnp.float32),
        ],
    )(csum2, cnt2, ewma_sum, ewma_n)


def kernel(x, vq, ewma_centroid_sum, ewma_centroid_n):
    xt = x.T
    x4 = xt.reshape(8, 8, 288, CHUNK).transpose(0, 2, 1, 3)
    x_flat = _sc_transpose(x4)
    idx3, quant_t = _dist_argmin(xt, vq, vq.T)
    idx2 = idx3.reshape(NW, NCH, CHUNK)
    zeros = jnp.zeros((NE, EMB), jnp.float32)
    zeros_n = jnp.zeros((NE, 16), jnp.float32)
    ones = jnp.ones((CHUNK, 16), jnp.float32)
    csum2, cnt2 = _sc_scatter(x_flat, idx2, zeros, zeros_n, ones)
    new_sum, new_n, new_vq = _ewma(
        csum2, cnt2, ewma_centroid_sum, ewma_centroid_n.reshape(NE, 1))
    return (quant_t.T, new_vq, new_sum, new_n.reshape(NE))


# speedup vs baseline: 2.3903x; 1.1288x over previous
import functools

import jax
import jax.numpy as jnp
from jax import lax
from jax.experimental import pallas as pl
from jax.experimental.pallas import tpu as pltpu
from jax.experimental.pallas import tpu_sc as plsc

EMB = 64
NE = 1024
NT = 36864
GAMMA = 0.99

NC = 2
NS = 16
NW = NC * NS
ROWS_PER_W = NT // NW
CHUNK = 128
NCH = ROWS_PER_W // CHUNK
SC_ROWS = NE // NS

BLK = ROWS_PER_W
NBLK = NT // BLK


def _dist_argmin_body(xt_ref, vq_ref, vqt_ref, idx_ref, quant_ref):
    xb = xt_ref[...]
    vqb = vq_ref[...]
    vtb = vqt_ref[...]
    dot = lax.dot_general(vqb, xb, (((1,), (0,)), ((), ())),
                          preferred_element_type=jnp.float32)
    xs = jnp.sum(xb * xb, axis=0, keepdims=True)
    vs = jnp.sum(vqb * vqb, axis=1, keepdims=True)
    d = xs - 2.0 * dot + vs
    idx = jnp.argmin(d, axis=0).astype(jnp.int32)
    iota = lax.broadcasted_iota(jnp.int32, d.shape, 0)
    idx_ref[0, 0] = idx
    onehot = (iota == idx[None, :]).astype(jnp.float32)
    quant_ref[...] = lax.dot_general(vtb, onehot, (((1,), (0,)), ((), ())),
                                     preferred_element_type=jnp.float32)


def _dist_argmin(xt, vq, vqt):
    return pl.pallas_call(
        _dist_argmin_body,
        grid=(NBLK,),
        in_specs=[
            pl.BlockSpec((EMB, BLK), lambda i: (0, i)),
            pl.BlockSpec((NE, EMB), lambda i: (0, 0)),
            pl.BlockSpec((EMB, NE), lambda i: (0, 0)),
        ],
        out_specs=[
            pl.BlockSpec((1, 1, BLK), lambda i: (i, 0, 0)),
            pl.BlockSpec((EMB, BLK), lambda i: (0, i)),
        ],
        out_shape=[
            jax.ShapeDtypeStruct((NBLK, 1, BLK), jnp.int32),
            jax.ShapeDtypeStruct((EMB, NT), jnp.float32),
        ],
    )(xt, vq, vqt)


def _sc_transpose(x4):
    mesh = plsc.VectorSubcoreMesh(core_axis_name="c", subcore_axis_name="s")
    half = ROWS_PER_W // 2

    @functools.partial(
        pl.kernel,
        out_type=jax.ShapeDtypeStruct((NT, EMB), jnp.float32),
        mesh=mesh,
        compiler_params=pltpu.CompilerParams(use_tc_tiling_on_sc=False,
                                             needs_layout_passes=False),
        scratch_types=[
            pltpu.VMEM((8, NCH, 8, CHUNK), jnp.float32),
            pltpu.VMEM((half, EMB + 1), jnp.float32),
        ],
    )
    def body(x4_hbm, xf_hbm, in_v, out_v):
        c = lax.axis_index("c")
        s = lax.axis_index("s")
        wid = s * NC + c
        base_t = wid * ROWS_PER_W
        lane = jax.lax.iota(jnp.int32, 16)
        pltpu.sync_copy(x4_hbm.at[:, pl.ds(wid * NCH, NCH)], in_v)
        for h in range(2):

            def body_d(d, carry):
                dhi = jnp.full((16,), d // 8, jnp.int32)
                dlo = jnp.full((16,), d % 8, jnp.int32)
                dv = jnp.full((16,), d, jnp.int32)
                for t0 in range(half // 16):
                    tl = h * half + t0 * 16
                    vals = plsc.load_gather(
                        in_v, [dhi, jnp.full((16,), tl // CHUNK, jnp.int32),
                               dlo, tl % CHUNK + lane])
                    plsc.store_scatter(out_v, [t0 * 16 + lane, dv], vals)
                return carry

            lax.fori_loop(0, EMB, body_d, 0)
            pltpu.sync_copy(out_v.at[:, pl.ds(0, EMB)],
                            xf_hbm.at[pl.ds(base_t + h * half, half)])

    return body(x4)


def _sc_scatter(x, idx2):
    mesh = plsc.VectorSubcoreMesh(core_axis_name="c", subcore_axis_name="s")

    @functools.partial(
        pl.kernel,
        out_type=[
            jax.ShapeDtypeStruct((NC, EMB, NE), jnp.float32),
            jax.ShapeDtypeStruct((NC, 16, NE), jnp.float32),
        ],
        mesh=mesh,
        compiler_params=pltpu.CompilerParams(use_tc_tiling_on_sc=False,
                                             needs_layout_passes=False),
        scratch_types=[
            pltpu.VMEM((NCH, CHUNK), jnp.int32),
            pltpu.VMEM((ROWS_PER_W, EMB), jnp.float32),
            pltpu.VMEM((CHUNK, 16), jnp.float32),
            pltpu.VMEM((SC_ROWS, EMB), jnp.float32),
            pltpu.VMEM((EMB, SC_ROWS + 1), jnp.float32),
            pltpu.VMEM_SHARED((NE, EMB), jnp.float32),
            pltpu.VMEM_SHARED((NE, 16), jnp.float32),
        ],
    )
    def body(x_hbm, idx_hbm, csumt_hbm, cntt_hbm,
             idx_v, buf_v, ones_v, pub_v, pubt_v, shared, shared_n):
        c = lax.axis_index("c")
        s = lax.axis_index("s")
        wid = s * NC + c
        lane = jax.lax.iota(jnp.int32, 16)
        zeros16 = jnp.zeros((16,), jnp.float32)
        ones16 = jnp.ones((16,), jnp.float32)
        for r in range(SC_ROWS):
            for q in range(EMB // 16):
                pub_v[r, pl.ds(q * 16, 16)] = zeros16
        for r in range(CHUNK):
            ones_v[r, pl.ds(0, 16)] = ones16
        pltpu.sync_copy(pub_v, shared.at[pl.ds(s * SC_ROWS, SC_ROWS)])
        pltpu.sync_copy(pub_v.at[pl.ds(0, SC_ROWS), pl.ds(0, 16)],
                        shared_n.at[pl.ds(s * SC_ROWS, SC_ROWS)])
        pltpu.sync_copy(idx_hbm.at[wid], idx_v)
        pltpu.sync_copy(x_hbm.at[pl.ds(wid * ROWS_PER_W, ROWS_PER_W)], buf_v)
        plsc.subcore_barrier()
        for j in range(NCH):
            pltpu.sync_copy(buf_v.at[pl.ds(j * CHUNK, CHUNK)],
                            shared.at[idx_v.at[j]], add=True)
            pltpu.sync_copy(ones_v, shared_n.at[idx_v.at[j]], add=True)
        plsc.subcore_barrier()
        pltpu.sync_copy(shared.at[pl.ds(s * SC_ROWS, SC_ROWS)], pub_v)

        def trans_row(r, carry):
            rvec = jnp.full((16,), r, jnp.int32)
            for q in range(EMB // 16):
                vals = plsc.load_gather(pub_v, [rvec, q * 16 + lane])
                plsc.store_scatter(pubt_v, [q * 16 + lane,
                                            jnp.full((16,), r, jnp.int32)],
                                   vals)
            return carry

        lax.fori_loop(0, SC_ROWS, trans_row, 0)
        pltpu.sync_copy(pubt_v.at[:, pl.ds(0, SC_ROWS)],
                        csumt_hbm.at[c, :, pl.ds(s * SC_ROWS, SC_ROWS)])
        pltpu.sync_copy(shared_n.at[pl.ds(s * SC_ROWS, SC_ROWS)],
                        pub_v.at[pl.ds(0, SC_ROWS), pl.ds(0, 16)])

        def trans_cnt(r, carry):
            rvec = jnp.full((16,), r, jnp.int32)
            vals = plsc.load_gather(pub_v, [rvec, lane])
            plsc.store_scatter(pubt_v, [lane, rvec], vals)
            return carry

        lax.fori_loop(0, SC_ROWS, trans_cnt, 0)
        pltpu.sync_copy(pubt_v.at[pl.ds(0, 16), pl.ds(0, SC_ROWS)],
                        cntt_hbm.at[c, :, pl.ds(s * SC_ROWS, SC_ROWS)])

    return body(x, idx2)


def _ewma_body(cs_ref, cnt_ref, es_ref, en_ref, ns_ref, nn_ref, nvq_ref):
    cs = cs_ref[0] + cs_ref[1]
    cnt = cnt_ref[0, 0:1, :] + cnt_ref[1, 0:1, :]
    new_sum = es_ref[...] * GAMMA + cs * (1.0 - GAMMA)
    new_n = en_ref[...] * GAMMA + cnt * (1.0 - GAMMA)
    ns_ref[...] = new_sum
    nn_ref[...] = new_n
    nvq_ref[...] = new_sum / new_n


def _ewma(csumt2, cntt2, ewma_sum_t, ewma_n_row):
    return pl.pallas_call(
        _ewma_body,
        out_shape=[
            jax.ShapeDtypeStruct((EMB, NE), jnp.float32),
            jax.ShapeDtypeStruct((1, NE), jnp.float32),
            jax.ShapeDtypeStruct((EMB, NE), jnp.float32),
        ],
    )(csumt2, cntt2, ewma_sum_t, ewma_n_row)


def kernel(x, vq, ewma_centroid_sum, ewma_centroid_n):
    xt = x.T
    x4 = xt.reshape(8, 8, 288, CHUNK).transpose(0, 2, 1, 3)
    x_flat = _sc_transpose(x4)
    idx3, quant_t = _dist_argmin(xt, vq, vq.T)
    idx2 = idx3.reshape(NW, NCH, CHUNK)
    csumt2, cntt2 = _sc_scatter(x_flat, idx2)
    new_sum_t, new_n1, new_vq_t = _ewma(
        csumt2, cntt2, ewma_centroid_sum.T, ewma_centroid_n.reshape(1, NE))
    return (quant_t.T, new_vq_t.T, new_sum_t.T, new_n1.reshape(NE))


# speedup vs baseline: 2.4975x; 1.0449x over previous
import functools

import jax
import jax.numpy as jnp
from jax import lax
from jax.experimental import pallas as pl
from jax.experimental.pallas import tpu as pltpu
from jax.experimental.pallas import tpu_sc as plsc

EMB = 64
NE = 1024
NT = 36864
GAMMA = 0.99

NC = 2
NS = 16
NW = NC * NS
ROWS_PER_W = NT // NW
CHUNK = 128
NCH = ROWS_PER_W // CHUNK
SC_ROWS = NE // NS

BLK = 2304
NBLK = NT // BLK


def _dist_argmin_body(xt_ref, vq_ref, vqt_ref, idx_ref, quant_ref):
    xb = xt_ref[...]
    vqb = vq_ref[...]
    vtb = vqt_ref[...]
    dot = lax.dot_general(vqb, xb, (((1,), (0,)), ((), ())),
                          preferred_element_type=jnp.float32)
    xs = jnp.sum(xb * xb, axis=0, keepdims=True)
    vs = jnp.sum(vqb * vqb, axis=1, keepdims=True)
    d = xs - 2.0 * dot + vs
    idx = jnp.argmin(d, axis=0).astype(jnp.int32)
    iota = lax.broadcasted_iota(jnp.int32, d.shape, 0)
    idx_ref[0, 0] = idx
    onehot = (iota == idx[None, :]).astype(jnp.float32)
    quant_ref[...] = lax.dot_general(vtb, onehot, (((1,), (0,)), ((), ())),
                                     preferred_element_type=jnp.float32)


def _dist_argmin(xt, vq, vqt):
    return pl.pallas_call(
        _dist_argmin_body,
        grid=(NBLK,),
        in_specs=[
            pl.BlockSpec((EMB, BLK), lambda i: (0, i)),
            pl.BlockSpec((NE, EMB), lambda i: (0, 0)),
            pl.BlockSpec((EMB, NE), lambda i: (0, 0)),
        ],
        out_specs=[
            pl.BlockSpec((1, 1, BLK), lambda i: (i, 0, 0)),
            pl.BlockSpec((EMB, BLK), lambda i: (0, i)),
        ],
        out_shape=[
            jax.ShapeDtypeStruct((NBLK, 1, BLK), jnp.int32),
            jax.ShapeDtypeStruct((EMB, NT), jnp.float32),
        ],
    )(xt, vq, vqt)


def _sc_transpose(x4):
    mesh = plsc.VectorSubcoreMesh(core_axis_name="c", subcore_axis_name="s")
    half = ROWS_PER_W // 2

    @functools.partial(
        pl.kernel,
        out_type=jax.ShapeDtypeStruct((NT, EMB), jnp.float32),
        mesh=mesh,
        compiler_params=pltpu.CompilerParams(use_tc_tiling_on_sc=False,
                                             needs_layout_passes=False),
        scratch_types=[
            pltpu.VMEM((8, NCH, 8, CHUNK), jnp.float32),
            pltpu.VMEM((half, EMB + 1), jnp.float32),
        ],
    )
    def body(x4_hbm, xf_hbm, in_v, out_v):
        c = lax.axis_index("c")
        s = lax.axis_index("s")
        wid = s * NC + c
        base_t = wid * ROWS_PER_W
        lane = jax.lax.iota(jnp.int32, 16)
        pltpu.sync_copy(x4_hbm.at[:, pl.ds(wid * NCH, NCH)], in_v)
        for h in range(2):

            def body_d(d, carry):
                dhi = jnp.full((16,), d // 8, jnp.int32)
                dlo = jnp.full((16,), d % 8, jnp.int32)
                dv = jnp.full((16,), d, jnp.int32)
                for t0 in range(half // 16):
                    tl = h * half + t0 * 16
                    vals = plsc.load_gather(
                        in_v, [dhi, jnp.full((16,), tl // CHUNK, jnp.int32),
                               dlo, tl % CHUNK + lane])
                    plsc.store_scatter(out_v, [t0 * 16 + lane, dv], vals)
                return carry

            lax.fori_loop(0, EMB, body_d, 0)
            pltpu.sync_copy(out_v.at[:, pl.ds(0, EMB)],
                            xf_hbm.at[pl.ds(base_t + h * half, half)])

    return body(x4)


def _sc_scatter(x, idx2):
    mesh = plsc.VectorSubcoreMesh(core_axis_name="c", subcore_axis_name="s")

    @functools.partial(
        pl.kernel,
        out_type=[
            jax.ShapeDtypeStruct((NC, EMB, NE), jnp.float32),
            jax.ShapeDtypeStruct((NC, 16, NE), jnp.float32),
        ],
        mesh=mesh,
        compiler_params=pltpu.CompilerParams(use_tc_tiling_on_sc=False,
                                             needs_layout_passes=False),
        scratch_types=[
            pltpu.VMEM((NCH, CHUNK), jnp.int32),
            pltpu.VMEM((ROWS_PER_W, EMB), jnp.float32),
            pltpu.VMEM((CHUNK, 16), jnp.float32),
            pltpu.VMEM((SC_ROWS, EMB), jnp.float32),
            pltpu.VMEM((EMB, SC_ROWS + 1), jnp.float32),
            pltpu.VMEM_SHARED((NE, EMB), jnp.float32),
            pltpu.VMEM_SHARED((NE, 16), jnp.float32),
        ],
    )
    def body(x_hbm, idx_hbm, csumt_hbm, cntt_hbm,
             idx_v, buf_v, ones_v, pub_v, pubt_v, shared, shared_n):
        c = lax.axis_index("c")
        s = lax.axis_index("s")
        wid = s * NC + c
        lane = jax.lax.iota(jnp.int32, 16)
        zeros16 = jnp.zeros((16,), jnp.float32)
        ones16 = jnp.ones((16,), jnp.float32)
        for r in range(SC_ROWS):
            for q in range(EMB // 16):
                pub_v[r, pl.ds(q * 16, 16)] = zeros16
        for r in range(CHUNK):
            ones_v[r, pl.ds(0, 16)] = ones16
        pltpu.sync_copy(pub_v, shared.at[pl.ds(s * SC_ROWS, SC_ROWS)])
        pltpu.sync_copy(pub_v.at[pl.ds(0, SC_ROWS), pl.ds(0, 16)],
                        shared_n.at[pl.ds(s * SC_ROWS, SC_ROWS)])
        pltpu.sync_copy(idx_hbm.at[wid], idx_v)
        pltpu.sync_copy(x_hbm.at[pl.ds(wid * ROWS_PER_W, ROWS_PER_W)], buf_v)
        plsc.subcore_barrier()
        for j in range(NCH):
            pltpu.sync_copy(buf_v.at[pl.ds(j * CHUNK, CHUNK)],
                            shared.at[idx_v.at[j]], add=True)
            pltpu.sync_copy(ones_v, shared_n.at[idx_v.at[j]], add=True)
        plsc.subcore_barrier()
        pltpu.sync_copy(shared.at[pl.ds(s * SC_ROWS, SC_ROWS)], pub_v)

        def trans_row(r, carry):
            rvec = jnp.full((16,), r, jnp.int32)
            for q in range(EMB // 16):
                vals = plsc.load_gather(pub_v, [rvec, q * 16 + lane])
                plsc.store_scatter(pubt_v, [q * 16 + lane,
                                            jnp.full((16,), r, jnp.int32)],
                                   vals)
            return carry

        lax.fori_loop(0, SC_ROWS, trans_row, 0)
        pltpu.sync_copy(pubt_v.at[:, pl.ds(0, SC_ROWS)],
                        csumt_hbm.at[c, :, pl.ds(s * SC_ROWS, SC_ROWS)])
        pltpu.sync_copy(shared_n.at[pl.ds(s * SC_ROWS, SC_ROWS)],
                        pub_v.at[pl.ds(0, SC_ROWS), pl.ds(0, 16)])

        def trans_cnt(r, carry):
            rvec = jnp.full((16,), r, jnp.int32)
            vals = plsc.load_gather(pub_v, [rvec, lane])
            plsc.store_scatter(pubt_v, [lane, rvec], vals)
            return carry

        lax.fori_loop(0, SC_ROWS, trans_cnt, 0)
        pltpu.sync_copy(pubt_v.at[pl.ds(0, 16), pl.ds(0, SC_ROWS)],
                        cntt_hbm.at[c, :, pl.ds(s * SC_ROWS, SC_ROWS)])

    return body(x, idx2)


def _ewma_body(cs_ref, cnt_ref, es_ref, en_ref, ns_ref, nn_ref, nvq_ref):
    cs = cs_ref[0] + cs_ref[1]
    cnt = cnt_ref[0, 0:1, :] + cnt_ref[1, 0:1, :]
    new_sum = es_ref[...] * GAMMA + cs * (1.0 - GAMMA)
    new_n = en_ref[...] * GAMMA + cnt * (1.0 - GAMMA)
    ns_ref[...] = new_sum
    nn_ref[...] = new_n
    nvq_ref[...] = new_sum / new_n


def _ewma(csumt2, cntt2, ewma_sum_t, ewma_n_row):
    return pl.pallas_call(
        _ewma_body,
        out_shape=[
            jax.ShapeDtypeStruct((EMB, NE), jnp.float32),
            jax.ShapeDtypeStruct((1, NE), jnp.float32),
            jax.ShapeDtypeStruct((EMB, NE), jnp.float32),
        ],
    )(csumt2, cntt2, ewma_sum_t, ewma_n_row)


def kernel(x, vq, ewma_centroid_sum, ewma_centroid_n):
    xt = x.T
    x4 = xt.reshape(8, 8, 288, CHUNK).transpose(0, 2, 1, 3)
    x_flat = _sc_transpose(x4)
    idx3, quant_t = _dist_argmin(xt, vq, vq.T)
    idx2 = idx3.reshape(NW, NCH, CHUNK)
    csumt2, cntt2 = _sc_scatter(x_flat, idx2)
    new_sum_t, new_n1, new_vq_t = _ewma(
        csumt2, cntt2, ewma_centroid_sum.T, ewma_centroid_n.reshape(1, NE))
    return (quant_t.T, new_vq_t.T, new_sum_t.T, new_n1.reshape(NE))
